# baseline (speedup 1.0000x reference)
import functools

import jax
import jax.numpy as jnp
from jax import lax
from jax.experimental import pallas as pl
from jax.experimental.pallas import tpu as pltpu
from jax.experimental.pallas import tpu_sc as plsc

_L = 2
_E = 8
_H = 768
_N = 4096
_RB = 512
_MB = 256
_NBLK = 40
_ROWS = _NBLK * _MB
_META = 64
_NW = 32
_TPW = _N // _NW


def _router_body(x_ref, rw_ref, rb_ref, e1_ref, e2_ref, r1_ref, r2_ref,
                 meta_ref, base_ref):
    i = pl.program_id(0)

    @pl.when(i == 0)
    def _():
        base_ref[...] = jnp.zeros((1, _E), jnp.float32)

    x = x_ref[...]
    logits = lax.dot_general(x, rw_ref[...], (((1,), (1,)), ((), ())))
    logits = logits + rb_ref[...]
    iota_e = lax.broadcasted_iota(jnp.int32, (_RB, _E), 1)
    m1 = jnp.max(logits, axis=1, keepdims=True)
    e1 = jnp.min(jnp.where(logits == m1, iota_e, _E), axis=1, keepdims=True)
    l2 = jnp.where(iota_e == e1, jnp.float32(-3.0e38), logits)
    m2 = jnp.max(l2, axis=1, keepdims=True)
    e2 = jnp.min(jnp.where(l2 == m2, iota_e, _E), axis=1, keepdims=True)
    oh1 = (iota_e == e1).astype(jnp.float32)
    oh2 = (iota_e == e2).astype(jnp.float32)
    ri = lax.broadcasted_iota(jnp.int32, (_RB, _RB), 0)
    ci = lax.broadcasted_iota(jnp.int32, (_RB, _RB), 1)
    ls = (ci < ri).astype(jnp.float32)
    rank1 = lax.dot_general(ls, oh1, (((1,), (0,)), ((), ())))
    rank2 = lax.dot_general(ls, oh2, (((1,), (0,)), ((), ())))
    base = base_ref[...]
    cs1 = jnp.sum(oh1, axis=0, keepdims=True)
    cs2 = jnp.sum(oh2, axis=0, keepdims=True)
    r1 = jnp.sum((rank1 + base) * oh1, axis=1)
    r2 = jnp.sum((rank2 + base + cs1) * oh2, axis=1)
    e1_ref[...] = e1[:, 0]
    e2_ref[...] = e2[:, 0]
    r1_ref[...] = r1.astype(jnp.int32)
    r2_ref[...] = r2.astype(jnp.int32)
    newbase = base + cs1 + cs2
    base_ref[...] = newbase

    @pl.when(i == pl.num_programs(0) - 1)
    def _():
        counts = newbase
        padded = jnp.floor((counts + (_MB - 1)) * (1.0 / _MB)) * _MB
        ei = lax.broadcasted_iota(jnp.int32, (_E, _E), 0)
        ej = lax.broadcasted_iota(jnp.int32, (_E, _E), 1)
        l8 = (ei < ej).astype(jnp.float32)
        pb = lax.dot_general(padded, l8, (((1,), (0,)), ((), ())))
        pbblk = pb * (1.0 / _MB)
        total_blk = (pb[:, _E - 1:_E] + padded[:, _E - 1:_E]) * (1.0 / _MB)
        li = lax.broadcasted_iota(jnp.int32, (1, _META), 1)
        lif = li.astype(jnp.float32)
        be = jnp.zeros((1, _META), jnp.float32)
        for e in range(_E):
            be = be + (lif >= pbblk[:, e:e + 1]).astype(jnp.float32)
        be = be - 1.0
        metaf = jnp.where(li < _NBLK, be, 0.0)
        for e in range(_E):
            metaf = metaf + pb[:, e:e + 1] * (li == 48 + e).astype(jnp.float32)
        metaf = metaf + total_blk * (li == 56).astype(jnp.float32)
        meta_ref[...] = jnp.reshape(metaf.astype(jnp.int32), (_META,))


def _router(x, rw, rb):
    return pl.pallas_call(
        _router_body,
        grid=(_N // _RB,),
        in_specs=[
            pl.BlockSpec((_RB, _H), lambda i: (i, 0)),
            pl.BlockSpec((_E, _H), lambda i: (0, 0)),
            pl.BlockSpec((1, _E), lambda i: (0, 0)),
        ],
        out_specs=[
            pl.BlockSpec((_RB,), lambda i: (i,)),
            pl.BlockSpec((_RB,), lambda i: (i,)),
            pl.BlockSpec((_RB,), lambda i: (i,)),
            pl.BlockSpec((_RB,), lambda i: (i,)),
            pl.BlockSpec((_META,), lambda i: (0,)),
        ],
        out_shape=[
            jax.ShapeDtypeStruct((_N,), jnp.int32),
            jax.ShapeDtypeStruct((_N,), jnp.int32),
            jax.ShapeDtypeStruct((_N,), jnp.int32),
            jax.ShapeDtypeStruct((_N,), jnp.int32),
            jax.ShapeDtypeStruct((_META,), jnp.int32),
        ],
        scratch_shapes=[pltpu.VMEM((1, _E), jnp.float32)],
    )(x, rw, rb.reshape(1, _E))


def _dispatch_body(e1_hbm, e2_hbm, r1_hbm, r2_hbm, meta_hbm, x_hbm,
                   p1_hbm, p2_hbm, xs_hbm, ev, rv, metav, pidx, rowbuf, sem):
    wid = lax.axis_index("s") * 2 + lax.axis_index("c")
    base = wid * _TPW
    pltpu.sync_copy(meta_hbm.at[pl.ds(48, 16)], metav)
    pbv = metav[...]
    gdn = lax.GatherDimensionNumbers(
        offset_dims=(), collapsed_slice_dims=(0,), start_index_map=(0,))
    for half, (ehbm, rhbm) in enumerate(((e1_hbm, r1_hbm), (e2_hbm, r2_hbm))):
        pltpu.sync_copy(ehbm.at[pl.ds(base, _TPW)], ev)
        pltpu.sync_copy(rhbm.at[pl.ds(base, _TPW)], rv)
        for i in range(_TPW // 16):
            e = ev[pl.ds(i * 16, 16)]
            r = rv[pl.ds(i * 16, 16)]
            pb_e = lax.gather(pbv, e[:, None], gdn, (1,),
                              mode=lax.GatherScatterMode.PROMISE_IN_BOUNDS)
            p = r + pb_e
            pidx[2 * half + i // 4, pl.ds((i % 4) * 16, 16)] = p
    pltpu.sync_copy(pidx.at[0], p1_hbm.at[pl.ds(base, 64)])
    pltpu.sync_copy(pidx.at[1], p1_hbm.at[pl.ds(base + 64, 64)])
    pltpu.sync_copy(pidx.at[2], p2_hbm.at[pl.ds(base, 64)])
    pltpu.sync_copy(pidx.at[3], p2_hbm.at[pl.ds(base + 64, 64)])
    for c in range(2):
        pltpu.sync_copy(x_hbm.at[pl.ds(base + c * 64, 64)], rowbuf)
        cp1 = pltpu.async_copy(rowbuf, xs_hbm.at[pidx.at[c]], sem)
        cp2 = pltpu.async_copy(rowbuf, xs_hbm.at[pidx.at[2 + c]], sem)
        cp1.wait()
        cp2.wait()


def _gmm_body(m_ref, x_ref, w_ref, b_ref, o_ref):
    i = pl.program_id(0)

    @pl.when(i < m_ref[56])
    def _():
        o_ref[...] = (
            lax.dot_general(x_ref[...], w_ref[0], (((1,), (1,)), ((), ())))
            + b_ref[0]
        )


def _gmm(meta, xs, ew, eb):
    gs = pltpu.PrefetchScalarGridSpec(
        num_scalar_prefetch=1,
        grid=(_NBLK,),
        in_specs=[
            pl.BlockSpec((_MB, _H), lambda i, m: (i, 0)),
            pl.BlockSpec((1, _H, _H), lambda i, m: (m[i], 0, 0)),
            pl.BlockSpec((1, 1, _H), lambda i, m: (m[i], 0, 0)),
        ],
        out_specs=pl.BlockSpec((_MB, _H), lambda i, m: (i, 0)),
    )
    return pl.pallas_call(
        _gmm_body,
        grid_spec=gs,
        out_shape=jax.ShapeDtypeStruct((_ROWS, _H), jnp.float32),
    )(meta, xs, ew, eb.reshape(_E, 1, _H))


def _combine_body(p1_hbm, p2_hbm, y_hbm, o_hbm, pv, qv, buf1, buf2, sem):
    wid = lax.axis_index("s") * 2 + lax.axis_index("c")
    base = wid * _TPW
    pltpu.sync_copy(p1_hbm.at[pl.ds(base, 64)], pv.at[0])
    pltpu.sync_copy(p1_hbm.at[pl.ds(base + 64, 64)], pv.at[1])
    pltpu.sync_copy(p2_hbm.at[pl.ds(base, 64)], qv.at[0])
    pltpu.sync_copy(p2_hbm.at[pl.ds(base + 64, 64)], qv.at[1])
    for c in range(2):
        g1 = pltpu.async_copy(y_hbm.at[pv.at[c]], buf1, sem)
        g2 = pltpu.async_copy(y_hbm.at[qv.at[c]], buf2, sem)
        g1.wait()
        g2.wait()

        def body(i, carry):
            for j in range(_H // 16):
                sl = pl.ds(j * 16, 16)
                buf1[i, sl] = buf1[i, sl] + buf2[i, sl]
            return carry

        lax.fori_loop(0, 64, body, 0)
        pltpu.sync_copy(buf1, o_hbm.at[pl.ds(base + c * 64, 64)])


@functools.lru_cache(maxsize=1)
def _sc_kernels():
    mesh = plsc.VectorSubcoreMesh(core_axis_name="c", subcore_axis_name="s")
    dispatch = functools.partial(
        pl.kernel,
        mesh=mesh,
        out_type=[
            jax.ShapeDtypeStruct((_N,), jnp.int32),
            jax.ShapeDtypeStruct((_N,), jnp.int32),
            jax.ShapeDtypeStruct((_ROWS, _H), jnp.float32),
        ],
        scratch_types=[
            pltpu.VMEM((_TPW,), jnp.int32),
            pltpu.VMEM((_TPW,), jnp.int32),
            pltpu.VMEM((16,), jnp.int32),
            pltpu.VMEM((4, 64), jnp.int32),
            pltpu.VMEM((64, _H), jnp.float32),
            pltpu.SemaphoreType.DMA,
        ],
    )(_dispatch_body)
    combine = functools.partial(
        pl.kernel,
        mesh=mesh,
        out_type=jax.ShapeDtypeStruct((_N, _H), jnp.float32),
        scratch_types=[
            pltpu.VMEM((2, 64), jnp.int32),
            pltpu.VMEM((2, 64), jnp.int32),
            pltpu.VMEM((64, _H), jnp.float32),
            pltpu.VMEM((64, _H), jnp.float32),
            pltpu.SemaphoreType.DMA,
        ],
    )(_combine_body)
    return dispatch, combine


def kernel(input_ids, router_w, router_b, expert_w, expert_b):
    bsz, seq = input_ids.shape
    hs = jax.random.normal(jax.random.key(42), (bsz, seq, _H), dtype=jnp.float32)
    x = hs.reshape(bsz * seq, _H)
    _dispatch, _combine = _sc_kernels()
    for l in range(_L):
        e1, e2, r1, r2, meta = _router(x, router_w[l], router_b[l])
        p1, p2, xs = _dispatch(e1, e2, r1, r2, meta, x)
        y = _gmm(meta, xs, expert_w[l], expert_b[l])
        x = _combine(p1, p2, y)
    return x.reshape(bsz, seq, _H)


# speedup vs baseline: 1.1533x; 1.1533x over previous
import functools

import jax
import jax.numpy as jnp
from jax import lax
from jax.experimental import pallas as pl
from jax.experimental.pallas import tpu as pltpu
from jax.experimental.pallas import tpu_sc as plsc

_L = 2
_E = 8
_H = 768
_N = 4096
_RB = 512
_MB = 256
_NBLK = 40
_ROWS = _NBLK * _MB
_META = 64
_NW = 32
_TPW = _N // _NW


def _router_body(x_ref, rw_ref, rb_ref, e1_ref, e2_ref, r1_ref, r2_ref,
                 meta_ref, base_ref):
    i = pl.program_id(0)

    @pl.when(i == 0)
    def _():
        base_ref[...] = jnp.zeros((_E, 1), jnp.float32)

    x = x_ref[...]
    lt = lax.dot_general(rw_ref[...], x, (((1,), (1,)), ((), ())))
    lt = lt + rb_ref[...]
    iota_s = lax.broadcasted_iota(jnp.int32, (_E, _RB), 0)
    m1 = jnp.max(lt, axis=0, keepdims=True)
    e1 = jnp.min(jnp.where(lt == m1, iota_s, _E), axis=0, keepdims=True)
    l2 = jnp.where(iota_s == e1, jnp.float32(-3.0e38), lt)
    m2 = jnp.max(l2, axis=0, keepdims=True)
    e2 = jnp.min(jnp.where(l2 == m2, iota_s, _E), axis=0, keepdims=True)
    oh1 = (iota_s == e1).astype(jnp.float32)
    oh2 = (iota_s == e2).astype(jnp.float32)
    def _excl_cumsum(v):
        k = 1
        while k < _RB:
            v = v + jnp.concatenate(
                [jnp.zeros((_E, k), v.dtype), v[:, :-k]], axis=1)
            k *= 2
        return v

    rank1 = _excl_cumsum(oh1) - oh1
    rank2 = _excl_cumsum(oh2) - oh2
    base = base_ref[...]
    cs1 = jnp.sum(oh1, axis=1, keepdims=True)
    cs2 = jnp.sum(oh2, axis=1, keepdims=True)
    r1 = jnp.sum((rank1 + base) * oh1, axis=0)
    r2 = jnp.sum((rank2 + base + cs1) * oh2, axis=0)
    e1_ref[...] = e1[0]
    e2_ref[...] = e2[0]
    r1_ref[...] = r1.astype(jnp.int32)
    r2_ref[...] = r2.astype(jnp.int32)
    newbase = base + cs1 + cs2
    base_ref[...] = newbase

    @pl.when(i == pl.num_programs(0) - 1)
    def _():
        counts = newbase
        padded = jnp.floor((counts + (_MB - 1)) * (1.0 / _MB)) * _MB
        ei = lax.broadcasted_iota(jnp.int32, (_E, _E), 0)
        ej = lax.broadcasted_iota(jnp.int32, (_E, _E), 1)
        l8 = (ej < ei).astype(jnp.float32)
        pb = lax.dot_general(l8, padded, (((1,), (0,)), ((), ())))
        pbblk = pb * (1.0 / _MB)
        total_blk = (pb[_E - 1:_E, :] + padded[_E - 1:_E, :]) * (1.0 / _MB)
        li = lax.broadcasted_iota(jnp.int32, (1, _META), 1)
        lif = li.astype(jnp.float32)
        be = jnp.zeros((1, _META), jnp.float32)
        for e in range(_E):
            be = be + (lif >= pbblk[e:e + 1, :]).astype(jnp.float32)
        be = be - 1.0
        metaf = jnp.where(li < _NBLK, be, 0.0)
        for e in range(_E):
            metaf = metaf + pb[e:e + 1, :] * (li == 48 + e).astype(jnp.float32)
        metaf = metaf + total_blk * (li == 56).astype(jnp.float32)
        meta_ref[...] = jnp.reshape(metaf.astype(jnp.int32), (_META,))


def _router(x, rw, rb):
    return pl.pallas_call(
        _router_body,
        grid=(_N // _RB,),
        in_specs=[
            pl.BlockSpec((_RB, _H), lambda i: (i, 0)),
            pl.BlockSpec((_E, _H), lambda i: (0, 0)),
            pl.BlockSpec((_E, 1), lambda i: (0, 0)),
        ],
        out_specs=[
            pl.BlockSpec((_RB,), lambda i: (i,)),
            pl.BlockSpec((_RB,), lambda i: (i,)),
            pl.BlockSpec((_RB,), lambda i: (i,)),
            pl.BlockSpec((_RB,), lambda i: (i,)),
            pl.BlockSpec((_META,), lambda i: (0,)),
        ],
        out_shape=[
            jax.ShapeDtypeStruct((_N,), jnp.int32),
            jax.ShapeDtypeStruct((_N,), jnp.int32),
            jax.ShapeDtypeStruct((_N,), jnp.int32),
            jax.ShapeDtypeStruct((_N,), jnp.int32),
            jax.ShapeDtypeStruct((_META,), jnp.int32),
        ],
        scratch_shapes=[pltpu.VMEM((_E, 1), jnp.float32)],
    )(x, rw, rb.reshape(_E, 1))


def _dispatch_body(e1_hbm, e2_hbm, r1_hbm, r2_hbm, meta_hbm, x_hbm,
                   p1_hbm, p2_hbm, xs_hbm, ev, rv, metav, pidx, rb0, rb1,
                   sem_l0, sem_l1, sem_st):
    wid = lax.axis_index("s") * 2 + lax.axis_index("c")
    base = wid * _TPW
    ld0 = pltpu.async_copy(x_hbm.at[pl.ds(base, 64)], rb0, sem_l0)
    ld1 = pltpu.async_copy(x_hbm.at[pl.ds(base + 64, 64)], rb1, sem_l1)
    pltpu.sync_copy(meta_hbm.at[pl.ds(48, 16)], metav)
    pbv = metav[...]
    gdn = lax.GatherDimensionNumbers(
        offset_dims=(), collapsed_slice_dims=(0,), start_index_map=(0,))
    for half, (ehbm, rhbm) in enumerate(((e1_hbm, r1_hbm), (e2_hbm, r2_hbm))):
        pltpu.sync_copy(ehbm.at[pl.ds(base, _TPW)], ev)
        pltpu.sync_copy(rhbm.at[pl.ds(base, _TPW)], rv)
        for i in range(_TPW // 16):
            e = ev[pl.ds(i * 16, 16)]
            r = rv[pl.ds(i * 16, 16)]
            pb_e = lax.gather(pbv, e[:, None], gdn, (1,),
                              mode=lax.GatherScatterMode.PROMISE_IN_BOUNDS)
            p = r + pb_e
            pidx[2 * half + i // 4, pl.ds((i % 4) * 16, 16)] = p
    ld0.wait()
    s0a = pltpu.async_copy(rb0, xs_hbm.at[pidx.at[0]], sem_st)
    s0b = pltpu.async_copy(rb0, xs_hbm.at[pidx.at[2]], sem_st)
    ld1.wait()
    s1a = pltpu.async_copy(rb1, xs_hbm.at[pidx.at[1]], sem_st)
    s1b = pltpu.async_copy(rb1, xs_hbm.at[pidx.at[3]], sem_st)
    pltpu.sync_copy(pidx.at[0], p1_hbm.at[pl.ds(base, 64)])
    pltpu.sync_copy(pidx.at[1], p1_hbm.at[pl.ds(base + 64, 64)])
    pltpu.sync_copy(pidx.at[2], p2_hbm.at[pl.ds(base, 64)])
    pltpu.sync_copy(pidx.at[3], p2_hbm.at[pl.ds(base + 64, 64)])
    s0a.wait()
    s0b.wait()
    s1a.wait()
    s1b.wait()


def _gmm_body(m_ref, x_ref, w_ref, b_ref, o_ref):
    i = pl.program_id(0)

    @pl.when(i < m_ref[56])
    def _():
        o_ref[...] = (
            lax.dot_general(x_ref[...], w_ref[0], (((1,), (1,)), ((), ())))
            + b_ref[0]
        )


def _gmm(meta, xs, ew, eb):
    gs = pltpu.PrefetchScalarGridSpec(
        num_scalar_prefetch=1,
        grid=(_NBLK,),
        in_specs=[
            pl.BlockSpec((_MB, _H), lambda i, m: (i, 0)),
            pl.BlockSpec((1, _H, _H), lambda i, m: (m[i], 0, 0)),
            pl.BlockSpec((1, 1, _H), lambda i, m: (m[i], 0, 0)),
        ],
        out_specs=pl.BlockSpec((_MB, _H), lambda i, m: (i, 0)),
    )
    return pl.pallas_call(
        _gmm_body,
        grid_spec=gs,
        out_shape=jax.ShapeDtypeStruct((_ROWS, _H), jnp.float32),
    )(meta, xs, ew, eb.reshape(_E, 1, _H))


def _combine_body(p1_hbm, p2_hbm, y_hbm, o_hbm, pv, qv, a0, b0, a1, b1,
                  sem_g0, sem_g1, sem_w0, sem_w1):
    wid = lax.axis_index("s") * 2 + lax.axis_index("c")
    base = wid * _TPW
    for c in range(4):
        pltpu.sync_copy(p1_hbm.at[pl.ds(base + 32 * c, 32)], pv.at[c])
        pltpu.sync_copy(p2_hbm.at[pl.ds(base + 32 * c, 32)], qv.at[c])
    bufs = ((a0, b0, sem_g0, sem_w0), (a1, b1, sem_g1, sem_w1))

    def _gather(c, a, b, sg):
        return (pltpu.async_copy(y_hbm.at[pv.at[c]], a, sg),
                pltpu.async_copy(y_hbm.at[qv.at[c]], b, sg))

    pend_g = [None, None]
    pend_s = [None, None]
    pend_g[0] = _gather(0, bufs[0][0], bufs[0][1], bufs[0][2])
    for c in range(4):
        s = c % 2
        o = (c + 1) % 2
        if c + 1 < 4:
            if pend_s[o] is not None:
                pend_s[o].wait()
            pend_g[o] = _gather(c + 1, bufs[o][0], bufs[o][1], bufs[o][2])
        g1, g2 = pend_g[s]
        g1.wait()
        g2.wait()
        a, b = bufs[s][0], bufs[s][1]

        def body(i, carry):
            for j in range(_H // 16):
                sl = pl.ds(j * 16, 16)
                a[i, sl] = a[i, sl] + b[i, sl]
            return carry

        lax.fori_loop(0, 32, body, 0)
        pend_s[s] = pltpu.async_copy(
            a, o_hbm.at[pl.ds(base + 32 * c, 32)], bufs[s][3])
    pend_s[0].wait()
    pend_s[1].wait()


@functools.lru_cache(maxsize=1)
def _sc_kernels():
    mesh = plsc.VectorSubcoreMesh(core_axis_name="c", subcore_axis_name="s")
    dispatch = functools.partial(
        pl.kernel,
        mesh=mesh,
        out_type=[
            jax.ShapeDtypeStruct((_N,), jnp.int32),
            jax.ShapeDtypeStruct((_N,), jnp.int32),
            jax.ShapeDtypeStruct((_ROWS, _H), jnp.float32),
        ],
        scratch_types=[
            pltpu.VMEM((_TPW,), jnp.int32),
            pltpu.VMEM((_TPW,), jnp.int32),
            pltpu.VMEM((16,), jnp.int32),
            pltpu.VMEM((4, 64), jnp.int32),
            pltpu.VMEM((64, _H), jnp.float32),
            pltpu.VMEM((64, _H), jnp.float32),
            pltpu.SemaphoreType.DMA,
            pltpu.SemaphoreType.DMA,
            pltpu.SemaphoreType.DMA,
        ],
    )(_dispatch_body)
    combine = functools.partial(
        pl.kernel,
        mesh=mesh,
        out_type=jax.ShapeDtypeStruct((_N, _H), jnp.float32),
        scratch_types=[
            pltpu.VMEM((4, 32), jnp.int32),
            pltpu.VMEM((4, 32), jnp.int32),
            pltpu.VMEM((32, _H), jnp.float32),
            pltpu.VMEM((32, _H), jnp.float32),
            pltpu.VMEM((32, _H), jnp.float32),
            pltpu.VMEM((32, _H), jnp.float32),
            pltpu.SemaphoreType.DMA,
            pltpu.SemaphoreType.DMA,
            pltpu.SemaphoreType.DMA,
            pltpu.SemaphoreType.DMA,
        ],
    )(_combine_body)
    return dispatch, combine


def kernel(input_ids, router_w, router_b, expert_w, expert_b):
    bsz, seq = input_ids.shape
    hs = jax.random.normal(jax.random.key(42), (bsz, seq, _H), dtype=jnp.float32)
    x = hs.reshape(bsz * seq, _H)
    _dispatch, _combine = _sc_kernels()
    for l in range(_L):
        e1, e2, r1, r2, meta = _router(x, router_w[l], router_b[l])
        p1, p2, xs = _dispatch(e1, e2, r1, r2, meta, x)
        y = _gmm(meta, xs, expert_w[l], expert_b[l])
        x = _combine(p1, p2, y)
    return x.reshape(bsz, seq, _H)


# speedup vs baseline: 1.1593x; 1.0052x over previous
import functools

import jax
import jax.numpy as jnp
from jax import lax
from jax.experimental import pallas as pl
from jax.experimental.pallas import tpu as pltpu
from jax.experimental.pallas import tpu_sc as plsc

_L = 2
_E = 8
_H = 768
_N = 4096
_RB = 512
_MB = 256
_NBLK = 40
_ROWS = _NBLK * _MB
_META = 64
_NW = 32
_TPW = _N // _NW


def _router_body(x_ref, rw_ref, rb_ref, e1_ref, e2_ref, r1_ref, r2_ref,
                 meta_ref, base_ref):
    i = pl.program_id(0)

    @pl.when(i == 0)
    def _():
        base_ref[...] = jnp.zeros((_E, 1), jnp.float32)

    x = x_ref[...]
    lt = lax.dot_general(rw_ref[...], x, (((1,), (1,)), ((), ())))
    lt = lt + rb_ref[...]
    iota_s = lax.broadcasted_iota(jnp.int32, (_E, _RB), 0)
    m1 = jnp.max(lt, axis=0, keepdims=True)
    e1 = jnp.min(jnp.where(lt == m1, iota_s, _E), axis=0, keepdims=True)
    l2 = jnp.where(iota_s == e1, jnp.float32(-3.0e38), lt)
    m2 = jnp.max(l2, axis=0, keepdims=True)
    e2 = jnp.min(jnp.where(l2 == m2, iota_s, _E), axis=0, keepdims=True)
    oh1 = (iota_s == e1).astype(jnp.float32)
    oh2 = (iota_s == e2).astype(jnp.float32)
    def _excl_cumsum(v):
        k = 1
        while k < _RB:
            v = v + jnp.concatenate(
                [jnp.zeros((_E, k), v.dtype), v[:, :-k]], axis=1)
            k *= 2
        return v

    rank1 = _excl_cumsum(oh1) - oh1
    rank2 = _excl_cumsum(oh2) - oh2
    base = base_ref[...]
    cs1 = jnp.sum(oh1, axis=1, keepdims=True)
    cs2 = jnp.sum(oh2, axis=1, keepdims=True)
    r1 = jnp.sum((rank1 + base) * oh1, axis=0)
    r2 = jnp.sum((rank2 + base + cs1) * oh2, axis=0)
    e1_ref[...] = e1[0]
    e2_ref[...] = e2[0]
    r1_ref[...] = r1.astype(jnp.int32)
    r2_ref[...] = r2.astype(jnp.int32)
    newbase = base + cs1 + cs2
    base_ref[...] = newbase

    @pl.when(i == pl.num_programs(0) - 1)
    def _():
        counts = newbase
        padded = jnp.floor((counts + (_MB - 1)) * (1.0 / _MB)) * _MB
        ei = lax.broadcasted_iota(jnp.int32, (_E, _E), 0)
        ej = lax.broadcasted_iota(jnp.int32, (_E, _E), 1)
        l8 = (ej < ei).astype(jnp.float32)
        pb = lax.dot_general(l8, padded, (((1,), (0,)), ((), ())))
        pbblk = pb * (1.0 / _MB)
        total_blk = (pb[_E - 1:_E, :] + padded[_E - 1:_E, :]) * (1.0 / _MB)
        li = lax.broadcasted_iota(jnp.int32, (1, _META), 1)
        lif = li.astype(jnp.float32)
        be = jnp.zeros((1, _META), jnp.float32)
        for e in range(_E):
            be = be + (lif >= pbblk[e:e + 1, :]).astype(jnp.float32)
        be = be - 1.0
        metaf = jnp.where(li < _NBLK, be, 0.0)
        for e in range(_E):
            metaf = metaf + pb[e:e + 1, :] * (li == 48 + e).astype(jnp.float32)
        metaf = metaf + total_blk * (li == 56).astype(jnp.float32)
        meta_ref[...] = jnp.reshape(metaf.astype(jnp.int32), (_META,))


def _router(x, rw, rb):
    return pl.pallas_call(
        _router_body,
        grid=(_N // _RB,),
        in_specs=[
            pl.BlockSpec((_RB, _H), lambda i: (i, 0)),
            pl.BlockSpec((_E, _H), lambda i: (0, 0)),
            pl.BlockSpec((_E, 1), lambda i: (0, 0)),
        ],
        out_specs=[
            pl.BlockSpec((_RB,), lambda i: (i,)),
            pl.BlockSpec((_RB,), lambda i: (i,)),
            pl.BlockSpec((_RB,), lambda i: (i,)),
            pl.BlockSpec((_RB,), lambda i: (i,)),
            pl.BlockSpec((_META,), lambda i: (0,)),
        ],
        out_shape=[
            jax.ShapeDtypeStruct((_N,), jnp.int32),
            jax.ShapeDtypeStruct((_N,), jnp.int32),
            jax.ShapeDtypeStruct((_N,), jnp.int32),
            jax.ShapeDtypeStruct((_N,), jnp.int32),
            jax.ShapeDtypeStruct((_META,), jnp.int32),
        ],
        scratch_shapes=[pltpu.VMEM((_E, 1), jnp.float32)],
    )(x, rw, rb.reshape(_E, 1))


def _dispatch_body(e1_hbm, e2_hbm, r1_hbm, r2_hbm, meta_hbm, x_hbm,
                   p1_hbm, p2_hbm, xs_hbm, ev, rv, metav, pidx, rb0, rb1,
                   sem_l0, sem_l1, sem_st):
    wid = lax.axis_index("s") * 2 + lax.axis_index("c")
    base = wid * _TPW
    ld0 = pltpu.async_copy(x_hbm.at[pl.ds(base, 64)], rb0, sem_l0)
    ld1 = pltpu.async_copy(x_hbm.at[pl.ds(base + 64, 64)], rb1, sem_l1)
    pltpu.sync_copy(meta_hbm.at[pl.ds(48, 16)], metav)
    pbv = metav[...]
    gdn = lax.GatherDimensionNumbers(
        offset_dims=(), collapsed_slice_dims=(0,), start_index_map=(0,))
    for half, (ehbm, rhbm) in enumerate(((e1_hbm, r1_hbm), (e2_hbm, r2_hbm))):
        pltpu.sync_copy(ehbm.at[pl.ds(base, _TPW)], ev)
        pltpu.sync_copy(rhbm.at[pl.ds(base, _TPW)], rv)
        for i in range(_TPW // 16):
            e = ev[pl.ds(i * 16, 16)]
            r = rv[pl.ds(i * 16, 16)]
            pb_e = lax.gather(pbv, e[:, None], gdn, (1,),
                              mode=lax.GatherScatterMode.PROMISE_IN_BOUNDS)
            p = r + pb_e
            pidx[2 * half + i // 4, pl.ds((i % 4) * 16, 16)] = p
    ld0.wait()
    s0a = pltpu.async_copy(rb0, xs_hbm.at[pidx.at[0]], sem_st)
    s0b = pltpu.async_copy(rb0, xs_hbm.at[pidx.at[2]], sem_st)
    ld1.wait()
    s1a = pltpu.async_copy(rb1, xs_hbm.at[pidx.at[1]], sem_st)
    s1b = pltpu.async_copy(rb1, xs_hbm.at[pidx.at[3]], sem_st)
    pltpu.sync_copy(pidx.at[0], p1_hbm.at[pl.ds(base, 64)])
    pltpu.sync_copy(pidx.at[1], p1_hbm.at[pl.ds(base + 64, 64)])
    pltpu.sync_copy(pidx.at[2], p2_hbm.at[pl.ds(base, 64)])
    pltpu.sync_copy(pidx.at[3], p2_hbm.at[pl.ds(base + 64, 64)])
    s0a.wait()
    s0b.wait()
    s1a.wait()
    s1b.wait()


def _gmm_body(m_ref, x_ref, w_ref, b_ref, o_ref):
    i = pl.program_id(0)

    @pl.when(i < m_ref[56])
    def _():
        e = m_ref[i]
        o_ref[...] = (
            lax.dot_general(x_ref[...], w_ref[e], (((1,), (1,)), ((), ())))
            + b_ref[e]
        )


def _gmm(meta, xs, ew, eb):
    gs = pltpu.PrefetchScalarGridSpec(
        num_scalar_prefetch=1,
        grid=(_NBLK,),
        in_specs=[
            pl.BlockSpec((_MB, _H), lambda i, m: (i, 0)),
            pl.BlockSpec((_E, _H, _H), lambda i, m: (0, 0, 0)),
            pl.BlockSpec((_E, 1, _H), lambda i, m: (0, 0, 0)),
        ],
        out_specs=pl.BlockSpec((_MB, _H), lambda i, m: (i, 0)),
    )
    return pl.pallas_call(
        _gmm_body,
        grid_spec=gs,
        out_shape=jax.ShapeDtypeStruct((_ROWS, _H), jnp.float32),
    )(meta, xs, ew, eb.reshape(_E, 1, _H))


def _combine_body(p1_hbm, p2_hbm, y_hbm, o_hbm, pv, qv, a0, b0, a1, b1,
                  sem_g0, sem_g1, sem_w0, sem_w1):
    wid = lax.axis_index("s") * 2 + lax.axis_index("c")
    base = wid * _TPW
    for c in range(4):
        pltpu.sync_copy(p1_hbm.at[pl.ds(base + 32 * c, 32)], pv.at[c])
        pltpu.sync_copy(p2_hbm.at[pl.ds(base + 32 * c, 32)], qv.at[c])
    bufs = ((a0, b0, sem_g0, sem_w0), (a1, b1, sem_g1, sem_w1))

    def _gather(c, a, b, sg):
        return (pltpu.async_copy(y_hbm.at[pv.at[c]], a, sg),
                pltpu.async_copy(y_hbm.at[qv.at[c]], b, sg))

    pend_g = [None, None]
    pend_s = [None, None]
    pend_g[0] = _gather(0, bufs[0][0], bufs[0][1], bufs[0][2])
    for c in range(4):
        s = c % 2
        o = (c + 1) % 2
        if c + 1 < 4:
            if pend_s[o] is not None:
                pend_s[o].wait()
            pend_g[o] = _gather(c + 1, bufs[o][0], bufs[o][1], bufs[o][2])
        g1, g2 = pend_g[s]
        g1.wait()
        g2.wait()
        a, b = bufs[s][0], bufs[s][1]

        def body(i, carry):
            for j in range(_H // 16):
                sl = pl.ds(j * 16, 16)
                a[i, sl] = a[i, sl] + b[i, sl]
            return carry

        lax.fori_loop(0, 32, body, 0)
        pend_s[s] = pltpu.async_copy(
            a, o_hbm.at[pl.ds(base + 32 * c, 32)], bufs[s][3])
    pend_s[0].wait()
    pend_s[1].wait()


@functools.lru_cache(maxsize=1)
def _sc_kernels():
    mesh = plsc.VectorSubcoreMesh(core_axis_name="c", subcore_axis_name="s")
    dispatch = functools.partial(
        pl.kernel,
        mesh=mesh,
        out_type=[
            jax.ShapeDtypeStruct((_N,), jnp.int32),
            jax.ShapeDtypeStruct((_N,), jnp.int32),
            jax.ShapeDtypeStruct((_ROWS, _H), jnp.float32),
        ],
        scratch_types=[
            pltpu.VMEM((_TPW,), jnp.int32),
            pltpu.VMEM((_TPW,), jnp.int32),
            pltpu.VMEM((16,), jnp.int32),
            pltpu.VMEM((4, 64), jnp.int32),
            pltpu.VMEM((64, _H), jnp.float32),
            pltpu.VMEM((64, _H), jnp.float32),
            pltpu.SemaphoreType.DMA,
            pltpu.SemaphoreType.DMA,
            pltpu.SemaphoreType.DMA,
        ],
    )(_dispatch_body)
    combine = functools.partial(
        pl.kernel,
        mesh=mesh,
        out_type=jax.ShapeDtypeStruct((_N, _H), jnp.float32),
        scratch_types=[
            pltpu.VMEM((4, 32), jnp.int32),
            pltpu.VMEM((4, 32), jnp.int32),
            pltpu.VMEM((32, _H), jnp.float32),
            pltpu.VMEM((32, _H), jnp.float32),
            pltpu.VMEM((32, _H), jnp.float32),
            pltpu.VMEM((32, _H), jnp.float32),
            pltpu.SemaphoreType.DMA,
            pltpu.SemaphoreType.DMA,
            pltpu.SemaphoreType.DMA,
            pltpu.SemaphoreType.DMA,
        ],
    )(_combine_body)
    return dispatch, combine


def kernel(input_ids, router_w, router_b, expert_w, expert_b):
    bsz, seq = input_ids.shape
    hs = jax.random.normal(jax.random.key(42), (bsz, seq, _H), dtype=jnp.float32)
    x = hs.reshape(bsz * seq, _H)
    _dispatch, _combine = _sc_kernels()
    for l in range(_L):
        e1, e2, r1, r2, meta = _router(x, router_w[l], router_b[l])
        p1, p2, xs = _dispatch(e1, e2, r1, r2, meta, x)
        y = _gmm(meta, xs, expert_w[l], expert_b[l])
        x = _combine(p1, p2, y)
    return x.reshape(bsz, seq, _H)


# speedup vs baseline: 1.2207x; 1.0530x over previous
import functools

import jax
import jax.numpy as jnp
from jax import lax
from jax.experimental import pallas as pl
from jax.experimental.pallas import tpu as pltpu
from jax.experimental.pallas import tpu_sc as plsc

_L = 2
_E = 8
_H = 768
_N = 4096
_RB = 512
_MB = 512
_NBLK = 23
_ROWS = _NBLK * _MB
_META = 64
_NW = 32
_TPW = _N // _NW


def _router_body(x_ref, rw_ref, rb_ref, e1_ref, e2_ref, r1_ref, r2_ref,
                 meta_ref, base_ref):
    i = pl.program_id(0)

    @pl.when(i == 0)
    def _():
        base_ref[...] = jnp.zeros((_E, 1), jnp.float32)

    x = x_ref[...]
    lt = lax.dot_general(rw_ref[...], x, (((1,), (1,)), ((), ())))
    lt = lt + rb_ref[...]
    iota_s = lax.broadcasted_iota(jnp.int32, (_E, _RB), 0)
    m1 = jnp.max(lt, axis=0, keepdims=True)
    e1 = jnp.min(jnp.where(lt == m1, iota_s, _E), axis=0, keepdims=True)
    l2 = jnp.where(iota_s == e1, jnp.float32(-3.0e38), lt)
    m2 = jnp.max(l2, axis=0, keepdims=True)
    e2 = jnp.min(jnp.where(l2 == m2, iota_s, _E), axis=0, keepdims=True)
    oh1 = (iota_s == e1).astype(jnp.float32)
    oh2 = (iota_s == e2).astype(jnp.float32)
    def _excl_cumsum(v):
        k = 1
        while k < _RB:
            v = v + jnp.concatenate(
                [jnp.zeros((_E, k), v.dtype), v[:, :-k]], axis=1)
            k *= 2
        return v

    rank1 = _excl_cumsum(oh1) - oh1
    rank2 = _excl_cumsum(oh2) - oh2
    base = base_ref[...]
    cs1 = jnp.sum(oh1, axis=1, keepdims=True)
    cs2 = jnp.sum(oh2, axis=1, keepdims=True)
    r1 = jnp.sum((rank1 + base) * oh1, axis=0)
    r2 = jnp.sum((rank2 + base + cs1) * oh2, axis=0)
    e1_ref[...] = e1[0]
    e2_ref[...] = e2[0]
    r1_ref[...] = r1.astype(jnp.int32)
    r2_ref[...] = r2.astype(jnp.int32)
    newbase = base + cs1 + cs2
    base_ref[...] = newbase

    @pl.when(i == pl.num_programs(0) - 1)
    def _():
        counts = newbase
        padded = jnp.floor((counts + (_MB - 1)) * (1.0 / _MB)) * _MB
        ei = lax.broadcasted_iota(jnp.int32, (_E, _E), 0)
        ej = lax.broadcasted_iota(jnp.int32, (_E, _E), 1)
        l8 = (ej < ei).astype(jnp.float32)
        pb = lax.dot_general(l8, padded, (((1,), (0,)), ((), ())))
        pbblk = pb * (1.0 / _MB)
        total_blk = (pb[_E - 1:_E, :] + padded[_E - 1:_E, :]) * (1.0 / _MB)
        li = lax.broadcasted_iota(jnp.int32, (1, _META), 1)
        lif = li.astype(jnp.float32)
        be = jnp.zeros((1, _META), jnp.float32)
        for e in range(_E):
            be = be + (lif >= pbblk[e:e + 1, :]).astype(jnp.float32)
        be = be - 1.0
        metaf = jnp.where(li < _NBLK, be, 0.0)
        for e in range(_E):
            metaf = metaf + pb[e:e + 1, :] * (li == 48 + e).astype(jnp.float32)
        metaf = metaf + total_blk * (li == 56).astype(jnp.float32)
        meta_ref[...] = jnp.reshape(metaf.astype(jnp.int32), (_META,))


def _router(x, rw, rb):
    return pl.pallas_call(
        _router_body,
        grid=(_N // _RB,),
        in_specs=[
            pl.BlockSpec((_RB, _H), lambda i: (i, 0)),
            pl.BlockSpec((_E, _H), lambda i: (0, 0)),
            pl.BlockSpec((_E, 1), lambda i: (0, 0)),
        ],
        out_specs=[
            pl.BlockSpec((_RB,), lambda i: (i,)),
            pl.BlockSpec((_RB,), lambda i: (i,)),
            pl.BlockSpec((_RB,), lambda i: (i,)),
            pl.BlockSpec((_RB,), lambda i: (i,)),
            pl.BlockSpec((_META,), lambda i: (0,)),
        ],
        out_shape=[
            jax.ShapeDtypeStruct((_N,), jnp.int32),
            jax.ShapeDtypeStruct((_N,), jnp.int32),
            jax.ShapeDtypeStruct((_N,), jnp.int32),
            jax.ShapeDtypeStruct((_N,), jnp.int32),
            jax.ShapeDtypeStruct((_META,), jnp.int32),
        ],
        scratch_shapes=[pltpu.VMEM((_E, 1), jnp.float32)],
    )(x, rw, rb.reshape(_E, 1))


def _dispatch_body(e1_hbm, e2_hbm, r1_hbm, r2_hbm, meta_hbm, x_hbm,
                   p1_hbm, p2_hbm, xs_hbm, ev, rv, metav, pidx, rb0, rb1,
                   sem_l0, sem_l1, sem_st):
    wid = lax.axis_index("s") * 2 + lax.axis_index("c")
    base = wid * _TPW
    ld0 = pltpu.async_copy(x_hbm.at[pl.ds(base, 64)], rb0, sem_l0)
    ld1 = pltpu.async_copy(x_hbm.at[pl.ds(base + 64, 64)], rb1, sem_l1)
    pltpu.sync_copy(meta_hbm.at[pl.ds(48, 16)], metav)
    pbv = metav[...]
    gdn = lax.GatherDimensionNumbers(
        offset_dims=(), collapsed_slice_dims=(0,), start_index_map=(0,))
    for half, (ehbm, rhbm) in enumerate(((e1_hbm, r1_hbm), (e2_hbm, r2_hbm))):
        pltpu.sync_copy(ehbm.at[pl.ds(base, _TPW)], ev)
        pltpu.sync_copy(rhbm.at[pl.ds(base, _TPW)], rv)
        for i in range(_TPW // 16):
            e = ev[pl.ds(i * 16, 16)]
            r = rv[pl.ds(i * 16, 16)]
            pb_e = lax.gather(pbv, e[:, None], gdn, (1,),
                              mode=lax.GatherScatterMode.PROMISE_IN_BOUNDS)
            p = r + pb_e
            pidx[2 * half + i // 4, pl.ds((i % 4) * 16, 16)] = p
    ld0.wait()
    s0a = pltpu.async_copy(rb0, xs_hbm.at[pidx.at[0]], sem_st)
    s0b = pltpu.async_copy(rb0, xs_hbm.at[pidx.at[2]], sem_st)
    ld1.wait()
    s1a = pltpu.async_copy(rb1, xs_hbm.at[pidx.at[1]], sem_st)
    s1b = pltpu.async_copy(rb1, xs_hbm.at[pidx.at[3]], sem_st)
    pltpu.sync_copy(pidx.at[0], p1_hbm.at[pl.ds(base, 64)])
    pltpu.sync_copy(pidx.at[1], p1_hbm.at[pl.ds(base + 64, 64)])
    pltpu.sync_copy(pidx.at[2], p2_hbm.at[pl.ds(base, 64)])
    pltpu.sync_copy(pidx.at[3], p2_hbm.at[pl.ds(base + 64, 64)])
    s0a.wait()
    s0b.wait()
    s1a.wait()
    s1b.wait()


def _gmm_body(m_ref, x_ref, w_ref, b_ref, o_ref):
    i = pl.program_id(0)

    @pl.when(i < m_ref[56])
    def _():
        e = m_ref[i]
        o_ref[...] = (
            lax.dot_general(x_ref[...], w_ref[e], (((1,), (1,)), ((), ())))
            + b_ref[e]
        )


def _gmm(meta, xs, ew, eb):
    gs = pltpu.PrefetchScalarGridSpec(
        num_scalar_prefetch=1,
        grid=(_NBLK,),
        in_specs=[
            pl.BlockSpec((_MB, _H), lambda i, m: (i, 0)),
            pl.BlockSpec((_E, _H, _H), lambda i, m: (0, 0, 0)),
            pl.BlockSpec((_E, 1, _H), lambda i, m: (0, 0, 0)),
        ],
        out_specs=pl.BlockSpec((_MB, _H), lambda i, m: (i, 0)),
    )
    return pl.pallas_call(
        _gmm_body,
        grid_spec=gs,
        out_shape=jax.ShapeDtypeStruct((_ROWS, _H), jnp.float32),
    )(meta, xs, ew, eb.reshape(_E, 1, _H))


def _combine_body(p1_hbm, p2_hbm, y_hbm, o_hbm, pv, qv, a0, b0, a1, b1,
                  sem_g0, sem_g1, sem_w0, sem_w1):
    wid = lax.axis_index("s") * 2 + lax.axis_index("c")
    base = wid * _TPW
    for c in range(4):
        pltpu.sync_copy(p1_hbm.at[pl.ds(base + 32 * c, 32)], pv.at[c])
        pltpu.sync_copy(p2_hbm.at[pl.ds(base + 32 * c, 32)], qv.at[c])
    bufs = ((a0, b0, sem_g0, sem_w0), (a1, b1, sem_g1, sem_w1))

    def _gather(c, a, b, sg):
        return (pltpu.async_copy(y_hbm.at[pv.at[c]], a, sg),
                pltpu.async_copy(y_hbm.at[qv.at[c]], b, sg))

    pend_g = [None, None]
    pend_s = [None, None]
    pend_g[0] = _gather(0, bufs[0][0], bufs[0][1], bufs[0][2])
    for c in range(4):
        s = c % 2
        o = (c + 1) % 2
        if c + 1 < 4:
            if pend_s[o] is not None:
                pend_s[o].wait()
            pend_g[o] = _gather(c + 1, bufs[o][0], bufs[o][1], bufs[o][2])
        g1, g2 = pend_g[s]
        g1.wait()
        g2.wait()
        a, b = bufs[s][0], bufs[s][1]

        def body(i, carry):
            for j in range(_H // 16):
                sl = pl.ds(j * 16, 16)
                a[i, sl] = a[i, sl] + b[i, sl]
            return carry

        lax.fori_loop(0, 32, body, 0)
        pend_s[s] = pltpu.async_copy(
            a, o_hbm.at[pl.ds(base + 32 * c, 32)], bufs[s][3])
    pend_s[0].wait()
    pend_s[1].wait()


@functools.lru_cache(maxsize=1)
def _sc_kernels():
    mesh = plsc.VectorSubcoreMesh(core_axis_name="c", subcore_axis_name="s")
    dispatch = functools.partial(
        pl.kernel,
        mesh=mesh,
        out_type=[
            jax.ShapeDtypeStruct((_N,), jnp.int32),
            jax.ShapeDtypeStruct((_N,), jnp.int32),
            jax.ShapeDtypeStruct((_ROWS, _H), jnp.float32),
        ],
        scratch_types=[
            pltpu.VMEM((_TPW,), jnp.int32),
            pltpu.VMEM((_TPW,), jnp.int32),
            pltpu.VMEM((16,), jnp.int32),
            pltpu.VMEM((4, 64), jnp.int32),
            pltpu.VMEM((64, _H), jnp.float32),
            pltpu.VMEM((64, _H), jnp.float32),
            pltpu.SemaphoreType.DMA,
            pltpu.SemaphoreType.DMA,
            pltpu.SemaphoreType.DMA,
        ],
    )(_dispatch_body)
    combine = functools.partial(
        pl.kernel,
        mesh=mesh,
        out_type=jax.ShapeDtypeStruct((_N, _H), jnp.float32),
        scratch_types=[
            pltpu.VMEM((4, 32), jnp.int32),
            pltpu.VMEM((4, 32), jnp.int32),
            pltpu.VMEM((32, _H), jnp.float32),
            pltpu.VMEM((32, _H), jnp.float32),
            pltpu.VMEM((32, _H), jnp.float32),
            pltpu.VMEM((32, _H), jnp.float32),
            pltpu.SemaphoreType.DMA,
            pltpu.SemaphoreType.DMA,
            pltpu.SemaphoreType.DMA,
            pltpu.SemaphoreType.DMA,
        ],
    )(_combine_body)
    return dispatch, combine


def kernel(input_ids, router_w, router_b, expert_w, expert_b):
    bsz, seq = input_ids.shape
    hs = jax.random.normal(jax.random.key(42), (bsz, seq, _H), dtype=jnp.float32)
    x = hs.reshape(bsz * seq, _H)
    _dispatch, _combine = _sc_kernels()
    for l in range(_L):
        e1, e2, r1, r2, meta = _router(x, router_w[l], router_b[l])
        p1, p2, xs = _dispatch(e1, e2, r1, r2, meta, x)
        y = _gmm(meta, xs, expert_w[l], expert_b[l])
        x = _combine(p1, p2, y)
    return x.reshape(bsz, seq, _H)


# speedup vs baseline: 1.3233x; 1.0841x over previous
import functools

import jax
import jax.numpy as jnp
from jax import lax
from jax.experimental import pallas as pl
from jax.experimental.pallas import tpu as pltpu
from jax.experimental.pallas import tpu_sc as plsc

_L = 2
_E = 8
_H = 768
_N = 4096
_RB = 512
_MB = 512
_NBLK = 23
_ROWS = _NBLK * _MB
_META = 64
_NW = 32
_TPW = _N // _NW


def _router_body(x_ref, rw_ref, rb_ref, e1_ref, e2_ref, r1_ref, r2_ref,
                 meta_ref, base_ref):
    i = pl.program_id(0)

    @pl.when(i == 0)
    def _():
        base_ref[...] = jnp.zeros((_E, 1), jnp.float32)

    x = x_ref[...]
    lt = lax.dot_general(rw_ref[0], x, (((1,), (1,)), ((), ())))
    lt = lt + rb_ref[0]
    iota_s = lax.broadcasted_iota(jnp.int32, (_E, _RB), 0)
    m1 = jnp.max(lt, axis=0, keepdims=True)
    e1 = jnp.min(jnp.where(lt == m1, iota_s, _E), axis=0, keepdims=True)
    l2 = jnp.where(iota_s == e1, jnp.float32(-3.0e38), lt)
    m2 = jnp.max(l2, axis=0, keepdims=True)
    e2 = jnp.min(jnp.where(l2 == m2, iota_s, _E), axis=0, keepdims=True)
    oh1 = (iota_s == e1).astype(jnp.float32)
    oh2 = (iota_s == e2).astype(jnp.float32)
    def _excl_cumsum(v):
        k = 1
        while k < _RB:
            v = v + jnp.concatenate(
                [jnp.zeros((_E, k), v.dtype), v[:, :-k]], axis=1)
            k *= 2
        return v

    rank1 = _excl_cumsum(oh1) - oh1
    rank2 = _excl_cumsum(oh2) - oh2
    base = base_ref[...]
    cs1 = jnp.sum(oh1, axis=1, keepdims=True)
    cs2 = jnp.sum(oh2, axis=1, keepdims=True)
    r1 = jnp.sum((rank1 + base) * oh1, axis=0)
    r2 = jnp.sum((rank2 + base + cs1) * oh2, axis=0)
    e1_ref[...] = e1[0]
    e2_ref[...] = e2[0]
    r1_ref[...] = r1.astype(jnp.int32)
    r2_ref[...] = r2.astype(jnp.int32)
    newbase = base + cs1 + cs2
    base_ref[...] = newbase

    @pl.when(i == pl.num_programs(0) - 1)
    def _():
        counts = newbase
        padded = jnp.floor((counts + (_MB - 1)) * (1.0 / _MB)) * _MB
        ei = lax.broadcasted_iota(jnp.int32, (_E, _E), 0)
        ej = lax.broadcasted_iota(jnp.int32, (_E, _E), 1)
        l8 = (ej < ei).astype(jnp.float32)
        pb = lax.dot_general(l8, padded, (((1,), (0,)), ((), ())))
        pbblk = pb * (1.0 / _MB)
        total_blk = (pb[_E - 1:_E, :] + padded[_E - 1:_E, :]) * (1.0 / _MB)
        li = lax.broadcasted_iota(jnp.int32, (1, _META), 1)
        lif = li.astype(jnp.float32)
        be = jnp.zeros((1, _META), jnp.float32)
        for e in range(_E):
            be = be + (lif >= pbblk[e:e + 1, :]).astype(jnp.float32)
        be = be - 1.0
        metaf = jnp.where(li < _NBLK, be, 0.0)
        for e in range(_E):
            metaf = metaf + pb[e:e + 1, :] * (li == 48 + e).astype(jnp.float32)
        metaf = metaf + total_blk * (li == 56).astype(jnp.float32)
        meta_ref[...] = jnp.reshape(metaf.astype(jnp.int32), (_META,))


def _router(x, rw, rb, l):
    return pl.pallas_call(
        _router_body,
        grid=(_N // _RB,),
        in_specs=[
            pl.BlockSpec((_RB, _H), lambda i: (i, 0)),
            pl.BlockSpec((1, _E, _H), lambda i, l=l: (l, 0, 0)),
            pl.BlockSpec((1, _E, 1), lambda i, l=l: (l, 0, 0)),
        ],
        out_specs=[
            pl.BlockSpec((_RB,), lambda i: (i,)),
            pl.BlockSpec((_RB,), lambda i: (i,)),
            pl.BlockSpec((_RB,), lambda i: (i,)),
            pl.BlockSpec((_RB,), lambda i: (i,)),
            pl.BlockSpec((_META,), lambda i: (0,)),
        ],
        out_shape=[
            jax.ShapeDtypeStruct((_N,), jnp.int32),
            jax.ShapeDtypeStruct((_N,), jnp.int32),
            jax.ShapeDtypeStruct((_N,), jnp.int32),
            jax.ShapeDtypeStruct((_N,), jnp.int32),
            jax.ShapeDtypeStruct((_META,), jnp.int32),
        ],
        scratch_shapes=[pltpu.VMEM((_E, 1), jnp.float32)],
    )(x, rw, rb.reshape(_L, _E, 1))


def _dispatch_body(e1_hbm, e2_hbm, r1_hbm, r2_hbm, meta_hbm, x_hbm,
                   p1_hbm, p2_hbm, xs_hbm, ev, rv, metav, pidx, rb0, rb1,
                   sem_l0, sem_l1, sem_st):
    wid = lax.axis_index("s") * 2 + lax.axis_index("c")
    base = wid * _TPW
    ld0 = pltpu.async_copy(x_hbm.at[pl.ds(base, 64)], rb0, sem_l0)
    ld1 = pltpu.async_copy(x_hbm.at[pl.ds(base + 64, 64)], rb1, sem_l1)
    pltpu.sync_copy(meta_hbm.at[pl.ds(48, 16)], metav)
    pbv = metav[...]
    gdn = lax.GatherDimensionNumbers(
        offset_dims=(), collapsed_slice_dims=(0,), start_index_map=(0,))
    for half, (ehbm, rhbm) in enumerate(((e1_hbm, r1_hbm), (e2_hbm, r2_hbm))):
        pltpu.sync_copy(ehbm.at[pl.ds(base, _TPW)], ev)
        pltpu.sync_copy(rhbm.at[pl.ds(base, _TPW)], rv)
        for i in range(_TPW // 16):
            e = ev[pl.ds(i * 16, 16)]
            r = rv[pl.ds(i * 16, 16)]
            pb_e = lax.gather(pbv, e[:, None], gdn, (1,),
                              mode=lax.GatherScatterMode.PROMISE_IN_BOUNDS)
            p = r + pb_e
            pidx[2 * half + i // 4, pl.ds((i % 4) * 16, 16)] = p
    ld0.wait()
    s0a = pltpu.async_copy(rb0, xs_hbm.at[pidx.at[0]], sem_st)
    s0b = pltpu.async_copy(rb0, xs_hbm.at[pidx.at[2]], sem_st)
    ld1.wait()
    s1a = pltpu.async_copy(rb1, xs_hbm.at[pidx.at[1]], sem_st)
    s1b = pltpu.async_copy(rb1, xs_hbm.at[pidx.at[3]], sem_st)
    pltpu.sync_copy(pidx.at[0], p1_hbm.at[pl.ds(base, 64)])
    pltpu.sync_copy(pidx.at[1], p1_hbm.at[pl.ds(base + 64, 64)])
    pltpu.sync_copy(pidx.at[2], p2_hbm.at[pl.ds(base, 64)])
    pltpu.sync_copy(pidx.at[3], p2_hbm.at[pl.ds(base + 64, 64)])
    s0a.wait()
    s0b.wait()
    s1a.wait()
    s1b.wait()


def _gmm_body(m_ref, x_ref, w_ref, b_ref, o_ref):
    i = pl.program_id(0)

    @pl.when(i < m_ref[56])
    def _():
        e = m_ref[i]
        o_ref[...] = (
            lax.dot_general(x_ref[...], w_ref[0, e], (((1,), (1,)), ((), ())))
            + b_ref[0, e]
        )


def _gmm(meta, xs, ew, eb, l):
    gs = pltpu.PrefetchScalarGridSpec(
        num_scalar_prefetch=1,
        grid=(_NBLK,),
        in_specs=[
            pl.BlockSpec((_MB, _H), lambda i, m: (i, 0)),
            pl.BlockSpec((1, _E, _H, _H), lambda i, m, l=l: (l, 0, 0, 0)),
            pl.BlockSpec((1, _E, 1, _H), lambda i, m, l=l: (l, 0, 0, 0)),
        ],
        out_specs=pl.BlockSpec((_MB, _H), lambda i, m: (i, 0)),
    )
    return pl.pallas_call(
        _gmm_body,
        grid_spec=gs,
        out_shape=jax.ShapeDtypeStruct((_ROWS, _H), jnp.float32),
    )(meta, xs, ew, eb.reshape(_L, _E, 1, _H))


def _combine_body(p1_hbm, p2_hbm, y_hbm, o_hbm, pv, qv, a0, b0, a1, b1,
                  sem_g0, sem_g1, sem_w0, sem_w1):
    wid = lax.axis_index("s") * 2 + lax.axis_index("c")
    base = wid * _TPW
    for c in range(4):
        pltpu.sync_copy(p1_hbm.at[pl.ds(base + 32 * c, 32)], pv.at[c])
        pltpu.sync_copy(p2_hbm.at[pl.ds(base + 32 * c, 32)], qv.at[c])
    bufs = ((a0, b0, sem_g0, sem_w0), (a1, b1, sem_g1, sem_w1))

    def _gather(c, a, b, sg):
        return (pltpu.async_copy(y_hbm.at[pv.at[c]], a, sg),
                pltpu.async_copy(y_hbm.at[qv.at[c]], b, sg))

    pend_g = [None, None]
    pend_s = [None, None]
    pend_g[0] = _gather(0, bufs[0][0], bufs[0][1], bufs[0][2])
    for c in range(4):
        s = c % 2
        o = (c + 1) % 2
        if c + 1 < 4:
            if pend_s[o] is not None:
                pend_s[o].wait()
            pend_g[o] = _gather(c + 1, bufs[o][0], bufs[o][1], bufs[o][2])
        g1, g2 = pend_g[s]
        g1.wait()
        g2.wait()
        a, b = bufs[s][0], bufs[s][1]

        def body(i, carry):
            for j in range(_H // 16):
                sl = pl.ds(j * 16, 16)
                a[i, sl] = a[i, sl] + b[i, sl]
            return carry

        lax.fori_loop(0, 32, body, 0)
        pend_s[s] = pltpu.async_copy(
            a, o_hbm.at[pl.ds(base + 32 * c, 32)], bufs[s][3])
    pend_s[0].wait()
    pend_s[1].wait()


@functools.lru_cache(maxsize=1)
def _sc_kernels():
    mesh = plsc.VectorSubcoreMesh(core_axis_name="c", subcore_axis_name="s")
    dispatch = functools.partial(
        pl.kernel,
        mesh=mesh,
        out_type=[
            jax.ShapeDtypeStruct((_N,), jnp.int32),
            jax.ShapeDtypeStruct((_N,), jnp.int32),
            jax.ShapeDtypeStruct((_ROWS, _H), jnp.float32),
        ],
        scratch_types=[
            pltpu.VMEM((_TPW,), jnp.int32),
            pltpu.VMEM((_TPW,), jnp.int32),
            pltpu.VMEM((16,), jnp.int32),
            pltpu.VMEM((4, 64), jnp.int32),
            pltpu.VMEM((64, _H), jnp.float32),
            pltpu.VMEM((64, _H), jnp.float32),
            pltpu.SemaphoreType.DMA,
            pltpu.SemaphoreType.DMA,
            pltpu.SemaphoreType.DMA,
        ],
    )(_dispatch_body)
    combine = functools.partial(
        pl.kernel,
        mesh=mesh,
        out_type=jax.ShapeDtypeStruct((_N, _H), jnp.float32),
        scratch_types=[
            pltpu.VMEM((4, 32), jnp.int32),
            pltpu.VMEM((4, 32), jnp.int32),
            pltpu.VMEM((32, _H), jnp.float32),
            pltpu.VMEM((32, _H), jnp.float32),
            pltpu.VMEM((32, _H), jnp.float32),
            pltpu.VMEM((32, _H), jnp.float32),
            pltpu.SemaphoreType.DMA,
            pltpu.SemaphoreType.DMA,
            pltpu.SemaphoreType.DMA,
            pltpu.SemaphoreType.DMA,
        ],
    )(_combine_body)
    return dispatch, combine


def kernel(input_ids, router_w, router_b, expert_w, expert_b):
    bsz, seq = input_ids.shape
    hs = jax.random.normal(jax.random.key(42), (bsz, seq, _H), dtype=jnp.float32)
    x = hs.reshape(bsz * seq, _H)
    _dispatch, _combine = _sc_kernels()
    for l in range(_L):
        e1, e2, r1, r2, meta = _router(x, router_w, router_b, l)
        p1, p2, xs = _dispatch(e1, e2, r1, r2, meta, x)
        y = _gmm(meta, xs, expert_w, expert_b, l)
        x = _combine(p1, p2, y)
    return x.reshape(bsz, seq, _H)
